# initial kernel scaffold (unmeasured)
import jax
import jax.numpy as jnp
from jax import lax
from jax.experimental import pallas as pl
from jax.experimental.pallas import tpu as pltpu

N_DEV = 8
M, K, N = 4096, 4096, 8192
CH = M // N_DEV

_MESH = pl.DeviceIdType.MESH


def _ar_epilogue(partial):

    def body(part_ref, out_ref, comm, pbuf, obuf,
             pb_sems, ob_sem, rs_send_sems, rs_recv_sems,
             ag_send_sems, ag_recv_sems, rs_credit, ag_credit):
        d = lax.axis_index("i")
        left = lax.rem(d + (N_DEV - 1), N_DEV)
        right = lax.rem(d + 1, N_DEV)

        barrier = pltpu.get_barrier_semaphore()
        pl.semaphore_signal(barrier, inc=1, device_id=(left,),
                            device_id_type=_MESH)
        pl.semaphore_signal(barrier, inc=1, device_id=(right,),
                            device_id_type=_MESH)
        pl.semaphore_wait(barrier, 2)

        c0 = lax.rem(d + (N_DEV - 1), N_DEV)
        cp0 = pltpu.make_async_copy(
            part_ref.at[pl.ds(c0 * CH, CH)], comm.at[0], pb_sems.at[0])
        cp0.start()
        c1 = lax.rem(d + (N_DEV - 2), N_DEV)
        cp1 = pltpu.make_async_copy(
            part_ref.at[pl.ds(c1 * CH, CH)], pbuf.at[0], pb_sems.at[1])
        cp1.start()
        cp0.wait()
        pending_pb = cp1

        acc = None
        for s in range(N_DEV - 1):
            send_slot = s % 2
            recv_slot = (s + 1) % 2
            if s >= 1:
                pl.semaphore_wait(rs_credit, 1)
            rdma = pltpu.make_async_remote_copy(
                src_ref=comm.at[send_slot],
                dst_ref=comm.at[recv_slot],
                send_sem=rs_send_sems.at[s],
                recv_sem=rs_recv_sems.at[s],
                device_id=(right,),
                device_id_type=_MESH,
            )
            rdma.start()
            if s < N_DEV - 2:
                cn = lax.rem(d + (2 * N_DEV - 3 - s), N_DEV)
                cpn = pltpu.make_async_copy(
                    part_ref.at[pl.ds(cn * CH, CH)],
                    pbuf.at[(s + 1) % 2], pb_sems.at[(s + 1) % 2])
                cpn.start()
            else:
                cpn = None
            rdma.wait()
            if s <= N_DEV - 3:
                pl.semaphore_signal(rs_credit, inc=1, device_id=(left,),
                                    device_id_type=_MESH)
            pending_pb.wait()
            pending_pb = cpn
            val = (comm[recv_slot].astype(jnp.float32)
                   + pbuf[s % 2].astype(jnp.float32))
            if s < N_DEV - 2:
                comm[recv_slot] = val.astype(jnp.bfloat16)
            else:
                acc = val

        own = acc.astype(jnp.bfloat16)
        comm[0] = own
        y_own = own.astype(jnp.float32)
        gmax = jnp.max(jnp.maximum(y_own, 0.0))

        obuf[...] = y_own
        out_dma = pltpu.make_async_copy(
            obuf, out_ref.at[pl.ds(d * CH, CH)], ob_sem)
        out_dma.start()

        pl.semaphore_signal(ag_credit, inc=1, device_id=(left,),
                            device_id_type=_MESH)

        for s in range(N_DEV - 1):
            send_slot = s % 2
            recv_slot = (s + 1) % 2
            pl.semaphore_wait(ag_credit, 1)
            rdma = pltpu.make_async_remote_copy(
                src_ref=comm.at[send_slot],
                dst_ref=comm.at[recv_slot],
                send_sem=ag_send_sems.at[s],
                recv_sem=ag_recv_sems.at[s],
                device_id=(right,),
                device_id_type=_MESH,
            )
            rdma.start()
            rdma.wait()
            if s <= N_DEV - 3:
                pl.semaphore_signal(ag_credit, inc=1, device_id=(left,),
                                    device_id_type=_MESH)
            cidx = lax.rem(d + (2 * N_DEV - 1 - s), N_DEV)
            y = comm[recv_slot].astype(jnp.float32)
            gmax = jnp.maximum(gmax, jnp.max(jnp.maximum(y, 0.0)))
            out_dma.wait()
            obuf[...] = y
            out_dma = pltpu.make_async_copy(
                obuf, out_ref.at[pl.ds(cidx * CH, CH)], ob_sem)
            out_dma.start()
        out_dma.wait()

        scale = gmax / 127.0
        inv = jnp.where(gmax > 0.0, 127.0 / gmax, 0.0)
        for c in range(N_DEV):
            ld = pltpu.make_async_copy(
                out_ref.at[pl.ds(c * CH, CH)], obuf, ob_sem)
            ld.start()
            ld.wait()
            q = jnp.clip(jnp.round(jnp.maximum(obuf[...], 0.0) * inv),
                         0.0, 127.0)
            obuf[...] = q * scale
            st = pltpu.make_async_copy(
                obuf, out_ref.at[pl.ds(c * CH, CH)], ob_sem)
            st.start()
            st.wait()

    return pl.pallas_call(
        body,
        out_shape=jax.ShapeDtypeStruct((M, N), jnp.float32),
        in_specs=[pl.BlockSpec(memory_space=pltpu.ANY)],
        out_specs=pl.BlockSpec(memory_space=pltpu.ANY),
        scratch_shapes=[
            pltpu.VMEM((2, CH, N), jnp.bfloat16),
            pltpu.VMEM((2, CH, N), jnp.bfloat16),
            pltpu.VMEM((CH, N), jnp.float32),
            pltpu.SemaphoreType.DMA((2,)),
            pltpu.SemaphoreType.DMA,
            pltpu.SemaphoreType.DMA((N_DEV - 1,)),
            pltpu.SemaphoreType.DMA((N_DEV - 1,)),
            pltpu.SemaphoreType.DMA((N_DEV - 1,)),
            pltpu.SemaphoreType.DMA((N_DEV - 1,)),
            pltpu.SemaphoreType.REGULAR,
            pltpu.SemaphoreType.REGULAR,
        ],
        compiler_params=pltpu.CompilerParams(collective_id=0),
    )(partial)


def kernel(x, w_mat):
    partial = jnp.dot(
        x.astype(jnp.bfloat16), w_mat.astype(jnp.bfloat16),
        preferred_element_type=jnp.float32,
    ).astype(jnp.bfloat16)
    return _ar_epilogue(partial)


# baseline (device time: 1446471 ns/iter reference)
import jax
import jax.numpy as jnp
from jax import lax
from jax.experimental import pallas as pl
from jax.experimental.pallas import tpu as pltpu

N_DEV = 8
M, K, N = 4096, 4096, 8192
CH = M // N_DEV
TR = 128

_MESH = pl.DeviceIdType.MESH


def _ar_epilogue(partial):

    def body(part_ref, out_ref, comm, pbuf, amax_buf,
             pb_sem, ob_sem, rs_send_sems, rs_recv_sems,
             ag_send_sems, ag_recv_sems, bc_send_sems, bc_recv_sems,
             rs_credit, ag_credit):
        d = lax.axis_index("i")
        left = lax.rem(d + (N_DEV - 1), N_DEV)
        right = lax.rem(d + 1, N_DEV)

        barrier = pltpu.get_barrier_semaphore()
        pl.semaphore_signal(barrier, inc=1, device_id=(left,),
                            device_id_type=_MESH)
        pl.semaphore_signal(barrier, inc=1, device_id=(right,),
                            device_id_type=_MESH)
        pl.semaphore_wait(barrier, 2)

        c0 = lax.rem(d + (N_DEV - 1), N_DEV)
        cp0 = pltpu.make_async_copy(
            part_ref.at[pl.ds(c0 * CH, CH)], comm.at[0], pb_sem)
        cp0.start()
        cp0.wait()

        own_max = jnp.float32(0.0)
        for s in range(N_DEV - 1):
            send_slot = s % 2
            recv_slot = (s + 1) % 2
            if s >= 1:
                pl.semaphore_wait(rs_credit, 1)
            rdma = pltpu.make_async_remote_copy(
                src_ref=comm.at[send_slot],
                dst_ref=comm.at[recv_slot],
                send_sem=rs_send_sems.at[s],
                recv_sem=rs_recv_sems.at[s],
                device_id=(right,),
                device_id_type=_MESH,
            )
            rdma.start()
            cn = lax.rem(d + (2 * N_DEV - 2 - s), N_DEV)
            cpn = pltpu.make_async_copy(
                part_ref.at[pl.ds(cn * CH, CH)], pbuf, pb_sem)
            cpn.start()
            rdma.wait()
            if s <= N_DEV - 3:
                pl.semaphore_signal(rs_credit, inc=1, device_id=(left,),
                                    device_id_type=_MESH)
            cpn.wait()

            if s < N_DEV - 2:
                def add_tile(t, _):
                    sl = pl.ds(t * TR, TR)
                    val = (comm[recv_slot, sl, :].astype(jnp.float32)
                           + pbuf[sl, :].astype(jnp.float32))
                    comm[recv_slot, sl, :] = val.astype(jnp.bfloat16)
                    return 0
                lax.fori_loop(0, CH // TR, add_tile, 0)
            else:
                def final_tile(t, mx):
                    sl = pl.ds(t * TR, TR)
                    val = (comm[recv_slot, sl, :].astype(jnp.float32)
                           + pbuf[sl, :].astype(jnp.float32))
                    own = val.astype(jnp.bfloat16)
                    comm[0, sl, :] = own
                    v32 = own.astype(jnp.float32)
                    return jnp.maximum(mx, jnp.max(jnp.maximum(v32, 0.0)))
                own_max = lax.fori_loop(0, CH // TR, final_tile, own_max)

        amax_buf[d, :, :] = jnp.full((8, 128), own_max, dtype=jnp.float32)
        bcasts = []
        for k in range(1, N_DEV):
            j = lax.rem(d + k, N_DEV)
            bc = pltpu.make_async_remote_copy(
                src_ref=amax_buf.at[d],
                dst_ref=amax_buf.at[d],
                send_sem=bc_send_sems.at[k - 1],
                recv_sem=bc_recv_sems.at[k - 1],
                device_id=(j,),
                device_id_type=_MESH,
            )
            bc.start()
            bcasts.append(bc)
        for bc in bcasts:
            bc.wait_recv()
        for bc in bcasts:
            bc.wait_send()
        gmax = jnp.max(amax_buf[...])

        scale = gmax / 127.0
        inv = jnp.where(gmax > 0.0, 127.0 / gmax, 0.0)

        def quant_tile(t, _):
            sl = pl.ds(t * TR, TR)
            v = comm[0, sl, :].astype(jnp.float32)
            q = jnp.clip(jnp.round(jnp.maximum(v, 0.0) * inv), 0.0, 127.0)
            comm[0, sl, :] = (q * scale).astype(jnp.bfloat16)
            return 0
        lax.fori_loop(0, CH // TR, quant_tile, 0)

        od = pltpu.make_async_copy(
            comm.at[0], out_ref.at[pl.ds(d * CH, CH)], ob_sem)
        od.start()
        od.wait()

        pl.semaphore_signal(ag_credit, inc=1, device_id=(left,),
                            device_id_type=_MESH)

        for s in range(N_DEV - 1):
            send_slot = s % 2
            recv_slot = (s + 1) % 2
            pl.semaphore_wait(ag_credit, 1)
            rdma = pltpu.make_async_remote_copy(
                src_ref=comm.at[send_slot],
                dst_ref=comm.at[recv_slot],
                send_sem=ag_send_sems.at[s],
                recv_sem=ag_recv_sems.at[s],
                device_id=(right,),
                device_id_type=_MESH,
            )
            rdma.start()
            rdma.wait()
            cidx = lax.rem(d + (2 * N_DEV - 1 - s), N_DEV)
            st = pltpu.make_async_copy(
                comm.at[recv_slot], out_ref.at[pl.ds(cidx * CH, CH)], ob_sem)
            st.start()
            st.wait()
            if s <= N_DEV - 3:
                pl.semaphore_signal(ag_credit, inc=1, device_id=(left,),
                                    device_id_type=_MESH)

    return pl.pallas_call(
        body,
        out_shape=jax.ShapeDtypeStruct((M, N), jnp.bfloat16),
        in_specs=[pl.BlockSpec(memory_space=pl.ANY)],
        out_specs=pl.BlockSpec(memory_space=pl.ANY),
        scratch_shapes=[
            pltpu.VMEM((2, CH, N), jnp.bfloat16),
            pltpu.VMEM((CH, N), jnp.bfloat16),
            pltpu.VMEM((N_DEV, 8, 128), jnp.float32),
            pltpu.SemaphoreType.DMA,
            pltpu.SemaphoreType.DMA,
            pltpu.SemaphoreType.DMA((N_DEV - 1,)),
            pltpu.SemaphoreType.DMA((N_DEV - 1,)),
            pltpu.SemaphoreType.DMA((N_DEV - 1,)),
            pltpu.SemaphoreType.DMA((N_DEV - 1,)),
            pltpu.SemaphoreType.DMA((N_DEV - 1,)),
            pltpu.SemaphoreType.DMA((N_DEV - 1,)),
            pltpu.SemaphoreType.REGULAR,
            pltpu.SemaphoreType.REGULAR,
        ],
        compiler_params=pltpu.CompilerParams(
            collective_id=0,
            vmem_limit_bytes=48 * 1024 * 1024,
        ),
    )(partial)


def kernel(x, w_mat):
    partial = jnp.dot(
        x.astype(jnp.bfloat16), w_mat.astype(jnp.bfloat16),
        preferred_element_type=jnp.float32,
    ).astype(jnp.bfloat16)
    return _ar_epilogue(partial)


# device time: 815757 ns/iter; 1.7732x vs baseline; 1.7732x over previous
import jax
import jax.numpy as jnp
from jax import lax
from jax.experimental import pallas as pl
from jax.experimental.pallas import tpu as pltpu

N_DEV = 8
M, K, N = 4096, 4096, 8192
CH = M // N_DEV
HN = N // 2
TR = 128

_MESH = pl.DeviceIdType.MESH


def _ar_epilogue(partial):

    def body(part_ref, out_ref, commR, commL, pbR, pbL, amax_buf,
             pb_semR, pb_semL, ob_semR, ob_semL,
             rsR_send, rsR_recv, rsL_send, rsL_recv,
             agR_send, agR_recv, agL_send, agL_recv,
             bc_send_sems, bc_recv_sems,
             rs_creditR, rs_creditL, ag_creditR, ag_creditL):
        d = lax.axis_index("i")
        left = lax.rem(d + (N_DEV - 1), N_DEV)
        right = lax.rem(d + 1, N_DEV)

        barrier = pltpu.get_barrier_semaphore()
        pl.semaphore_signal(barrier, inc=1, device_id=(left,),
                            device_id_type=_MESH)
        pl.semaphore_signal(barrier, inc=1, device_id=(right,),
                            device_id_type=_MESH)
        pl.semaphore_wait(barrier, 2)

        cR0 = lax.rem(d + (N_DEV - 1), N_DEV)
        cL0 = lax.rem(d + 1, N_DEV)
        cp0 = pltpu.make_async_copy(
            part_ref.at[pl.ds(cR0 * CH, CH), pl.ds(0, HN)],
            commR.at[0], pb_semR)
        cp0.start()
        cp1 = pltpu.make_async_copy(
            part_ref.at[pl.ds(cL0 * CH, CH), pl.ds(HN, HN)],
            commL.at[0], pb_semL)
        cp1.start()
        cp0.wait()
        cp1.wait()

        own_max = jnp.float32(0.0)
        for s in range(N_DEV - 1):
            send_slot = s % 2
            recv_slot = (s + 1) % 2
            if s >= 1:
                pl.semaphore_wait(rs_creditR, 1)
                pl.semaphore_wait(rs_creditL, 1)
            rdmaR = pltpu.make_async_remote_copy(
                src_ref=commR.at[send_slot], dst_ref=commR.at[recv_slot],
                send_sem=rsR_send.at[s], recv_sem=rsR_recv.at[s],
                device_id=(right,), device_id_type=_MESH)
            rdmaR.start()
            rdmaL = pltpu.make_async_remote_copy(
                src_ref=commL.at[send_slot], dst_ref=commL.at[recv_slot],
                send_sem=rsL_send.at[s], recv_sem=rsL_recv.at[s],
                device_id=(left,), device_id_type=_MESH)
            rdmaL.start()
            cR = lax.rem(d + (2 * N_DEV - 2 - s), N_DEV)
            cL = lax.rem(d + 2 + s, N_DEV)
            dmaR = pltpu.make_async_copy(
                part_ref.at[pl.ds(cR * CH, CH), pl.ds(0, HN)], pbR, pb_semR)
            dmaR.start()
            dmaL = pltpu.make_async_copy(
                part_ref.at[pl.ds(cL * CH, CH), pl.ds(HN, HN)], pbL, pb_semL)
            dmaL.start()
            rdmaR.wait()
            rdmaL.wait()
            if s <= N_DEV - 3:
                pl.semaphore_signal(rs_creditR, inc=1, device_id=(left,),
                                    device_id_type=_MESH)
                pl.semaphore_signal(rs_creditL, inc=1, device_id=(right,),
                                    device_id_type=_MESH)
            dmaR.wait()
            dmaL.wait()

            if s < N_DEV - 2:
                def add_tile(t, _):
                    sl = pl.ds(t * TR, TR)
                    vR = (commR[recv_slot, sl, :].astype(jnp.float32)
                          + pbR[sl, :].astype(jnp.float32))
                    commR[recv_slot, sl, :] = vR.astype(jnp.bfloat16)
                    vL = (commL[recv_slot, sl, :].astype(jnp.float32)
                          + pbL[sl, :].astype(jnp.float32))
                    commL[recv_slot, sl, :] = vL.astype(jnp.bfloat16)
                    return 0
                lax.fori_loop(0, CH // TR, add_tile, 0)
            else:
                def final_tile(t, mx):
                    sl = pl.ds(t * TR, TR)
                    vR = (commR[recv_slot, sl, :].astype(jnp.float32)
                          + pbR[sl, :].astype(jnp.float32))
                    ownR = vR.astype(jnp.bfloat16)
                    commR[0, sl, :] = ownR
                    mx = jnp.maximum(mx, jnp.max(jnp.maximum(
                        ownR.astype(jnp.float32), 0.0)))
                    vL = (commL[recv_slot, sl, :].astype(jnp.float32)
                          + pbL[sl, :].astype(jnp.float32))
                    ownL = vL.astype(jnp.bfloat16)
                    commL[0, sl, :] = ownL
                    return jnp.maximum(mx, jnp.max(jnp.maximum(
                        ownL.astype(jnp.float32), 0.0)))
                own_max = lax.fori_loop(0, CH // TR, final_tile, own_max)

        amax_buf[d, :, :] = jnp.full((8, 128), own_max, dtype=jnp.float32)
        bcasts = []
        for k in range(1, N_DEV):
            j = lax.rem(d + k, N_DEV)
            bc = pltpu.make_async_remote_copy(
                src_ref=amax_buf.at[d],
                dst_ref=amax_buf.at[d],
                send_sem=bc_send_sems.at[k - 1],
                recv_sem=bc_recv_sems.at[k - 1],
                device_id=(j,),
                device_id_type=_MESH,
            )
            bc.start()
            bcasts.append(bc)
        for bc in bcasts:
            bc.wait_recv()
        for bc in bcasts:
            bc.wait_send()
        gmax = jnp.max(amax_buf[...])

        scale = gmax / 127.0
        inv = jnp.where(gmax > 0.0, 127.0 / gmax, 0.0)

        def quant_tile(t, _):
            sl = pl.ds(t * TR, TR)
            vR = commR[0, sl, :].astype(jnp.float32)
            qR = jnp.clip(jnp.round(jnp.maximum(vR, 0.0) * inv), 0.0, 127.0)
            commR[0, sl, :] = (qR * scale).astype(jnp.bfloat16)
            vL = commL[0, sl, :].astype(jnp.float32)
            qL = jnp.clip(jnp.round(jnp.maximum(vL, 0.0) * inv), 0.0, 127.0)
            commL[0, sl, :] = (qL * scale).astype(jnp.bfloat16)
            return 0
        lax.fori_loop(0, CH // TR, quant_tile, 0)

        odR = pltpu.make_async_copy(
            commR.at[0], out_ref.at[pl.ds(d * CH, CH), pl.ds(0, HN)], ob_semR)
        odR.start()
        odL = pltpu.make_async_copy(
            commL.at[0], out_ref.at[pl.ds(d * CH, CH), pl.ds(HN, HN)], ob_semL)
        odL.start()
        odR.wait()
        odL.wait()

        pl.semaphore_signal(ag_creditR, inc=1, device_id=(left,),
                            device_id_type=_MESH)
        pl.semaphore_signal(ag_creditL, inc=1, device_id=(right,),
                            device_id_type=_MESH)

        for s in range(N_DEV - 1):
            send_slot = s % 2
            recv_slot = (s + 1) % 2
            pl.semaphore_wait(ag_creditR, 1)
            pl.semaphore_wait(ag_creditL, 1)
            rdmaR = pltpu.make_async_remote_copy(
                src_ref=commR.at[send_slot], dst_ref=commR.at[recv_slot],
                send_sem=agR_send.at[s], recv_sem=agR_recv.at[s],
                device_id=(right,), device_id_type=_MESH)
            rdmaR.start()
            rdmaL = pltpu.make_async_remote_copy(
                src_ref=commL.at[send_slot], dst_ref=commL.at[recv_slot],
                send_sem=agL_send.at[s], recv_sem=agL_recv.at[s],
                device_id=(left,), device_id_type=_MESH)
            rdmaL.start()
            rdmaR.wait()
            rdmaL.wait()
            cR = lax.rem(d + (2 * N_DEV - 1 - s), N_DEV)
            cL = lax.rem(d + 1 + s, N_DEV)
            stR = pltpu.make_async_copy(
                commR.at[recv_slot],
                out_ref.at[pl.ds(cR * CH, CH), pl.ds(0, HN)], ob_semR)
            stR.start()
            stL = pltpu.make_async_copy(
                commL.at[recv_slot],
                out_ref.at[pl.ds(cL * CH, CH), pl.ds(HN, HN)], ob_semL)
            stL.start()
            stR.wait()
            stL.wait()
            if s <= N_DEV - 3:
                pl.semaphore_signal(ag_creditR, inc=1, device_id=(left,),
                                    device_id_type=_MESH)
                pl.semaphore_signal(ag_creditL, inc=1, device_id=(right,),
                                    device_id_type=_MESH)

    return pl.pallas_call(
        body,
        out_shape=jax.ShapeDtypeStruct((M, N), jnp.bfloat16),
        in_specs=[pl.BlockSpec(memory_space=pl.ANY)],
        out_specs=pl.BlockSpec(memory_space=pl.ANY),
        scratch_shapes=[
            pltpu.VMEM((2, CH, HN), jnp.bfloat16),
            pltpu.VMEM((2, CH, HN), jnp.bfloat16),
            pltpu.VMEM((CH, HN), jnp.bfloat16),
            pltpu.VMEM((CH, HN), jnp.bfloat16),
            pltpu.VMEM((N_DEV, 8, 128), jnp.float32),
            pltpu.SemaphoreType.DMA,
            pltpu.SemaphoreType.DMA,
            pltpu.SemaphoreType.DMA,
            pltpu.SemaphoreType.DMA,
            pltpu.SemaphoreType.DMA((N_DEV - 1,)),
            pltpu.SemaphoreType.DMA((N_DEV - 1,)),
            pltpu.SemaphoreType.DMA((N_DEV - 1,)),
            pltpu.SemaphoreType.DMA((N_DEV - 1,)),
            pltpu.SemaphoreType.DMA((N_DEV - 1,)),
            pltpu.SemaphoreType.DMA((N_DEV - 1,)),
            pltpu.SemaphoreType.DMA((N_DEV - 1,)),
            pltpu.SemaphoreType.DMA((N_DEV - 1,)),
            pltpu.SemaphoreType.DMA((N_DEV - 1,)),
            pltpu.SemaphoreType.DMA((N_DEV - 1,)),
            pltpu.SemaphoreType.REGULAR,
            pltpu.SemaphoreType.REGULAR,
            pltpu.SemaphoreType.REGULAR,
            pltpu.SemaphoreType.REGULAR,
        ],
        compiler_params=pltpu.CompilerParams(
            collective_id=0,
            vmem_limit_bytes=48 * 1024 * 1024,
        ),
    )(partial)


def kernel(x, w_mat):
    partial = jnp.dot(
        x.astype(jnp.bfloat16), w_mat.astype(jnp.bfloat16),
        preferred_element_type=jnp.float32,
    ).astype(jnp.bfloat16)
    return _ar_epilogue(partial)


# device time: 665668 ns/iter; 2.1730x vs baseline; 1.2255x over previous
import jax
import jax.numpy as jnp
from jax import lax
from jax.experimental import pallas as pl
from jax.experimental.pallas import tpu as pltpu

N_DEV = 8
M, K, N = 4096, 4096, 8192
CH = M // N_DEV
HN = N // 2
TR = 128

_MESH = pl.DeviceIdType.MESH


def _ar_epilogue(partial):

    def body(part_ref, out_ref, commR, commL, qcommR, qcommL,
             pbR, pbL, amax_buf,
             pb_semR, pb_semL, ob_semR, ob_semL,
             rsR_send, rsR_recv, rsL_send, rsL_recv,
             agR_send, agR_recv, agL_send, agL_recv,
             bc_send_sems, bc_recv_sems,
             rs_creditR, rs_creditL, ag_creditR, ag_creditL):
        d = lax.axis_index("i")
        left = lax.rem(d + (N_DEV - 1), N_DEV)
        right = lax.rem(d + 1, N_DEV)

        barrier = pltpu.get_barrier_semaphore()
        pl.semaphore_signal(barrier, inc=1, device_id=(left,),
                            device_id_type=_MESH)
        pl.semaphore_signal(barrier, inc=1, device_id=(right,),
                            device_id_type=_MESH)
        pl.semaphore_wait(barrier, 2)

        cR0 = lax.rem(d + (N_DEV - 1), N_DEV)
        cL0 = lax.rem(d + 1, N_DEV)
        cp0 = pltpu.make_async_copy(
            part_ref.at[pl.ds(cR0 * CH, CH), pl.ds(0, HN)],
            commR.at[0], pb_semR)
        cp0.start()
        cp1 = pltpu.make_async_copy(
            part_ref.at[pl.ds(cL0 * CH, CH), pl.ds(HN, HN)],
            commL.at[0], pb_semL)
        cp1.start()
        cp0.wait()
        cp1.wait()

        own_max = jnp.float32(0.0)
        for s in range(N_DEV - 1):
            send_slot = s % 2
            recv_slot = (s + 1) % 2
            if s >= 1:
                pl.semaphore_wait(rs_creditR, 1)
                pl.semaphore_wait(rs_creditL, 1)
            rdmaR = pltpu.make_async_remote_copy(
                src_ref=commR.at[send_slot], dst_ref=commR.at[recv_slot],
                send_sem=rsR_send.at[s], recv_sem=rsR_recv.at[s],
                device_id=(right,), device_id_type=_MESH)
            rdmaR.start()
            rdmaL = pltpu.make_async_remote_copy(
                src_ref=commL.at[send_slot], dst_ref=commL.at[recv_slot],
                send_sem=rsL_send.at[s], recv_sem=rsL_recv.at[s],
                device_id=(left,), device_id_type=_MESH)
            rdmaL.start()
            cR = lax.rem(d + (2 * N_DEV - 2 - s), N_DEV)
            cL = lax.rem(d + 2 + s, N_DEV)
            dmaR = pltpu.make_async_copy(
                part_ref.at[pl.ds(cR * CH, CH), pl.ds(0, HN)], pbR, pb_semR)
            dmaR.start()
            dmaL = pltpu.make_async_copy(
                part_ref.at[pl.ds(cL * CH, CH), pl.ds(HN, HN)], pbL, pb_semL)
            dmaL.start()
            rdmaR.wait()
            rdmaL.wait()
            if s <= N_DEV - 3:
                pl.semaphore_signal(rs_creditR, inc=1, device_id=(left,),
                                    device_id_type=_MESH)
                pl.semaphore_signal(rs_creditL, inc=1, device_id=(right,),
                                    device_id_type=_MESH)
            dmaR.wait()
            dmaL.wait()

            if s < N_DEV - 2:
                def add_tile(t, _):
                    sl = pl.ds(t * TR, TR)
                    vR = (commR[recv_slot, sl, :].astype(jnp.float32)
                          + pbR[sl, :].astype(jnp.float32))
                    commR[recv_slot, sl, :] = vR.astype(jnp.bfloat16)
                    vL = (commL[recv_slot, sl, :].astype(jnp.float32)
                          + pbL[sl, :].astype(jnp.float32))
                    commL[recv_slot, sl, :] = vL.astype(jnp.bfloat16)
                    return 0
                lax.fori_loop(0, CH // TR, add_tile, 0)
            else:
                def final_tile(t, mx):
                    sl = pl.ds(t * TR, TR)
                    vR = (commR[recv_slot, sl, :].astype(jnp.float32)
                          + pbR[sl, :].astype(jnp.float32))
                    ownR = vR.astype(jnp.bfloat16)
                    commR[0, sl, :] = ownR
                    mx = jnp.maximum(mx, jnp.max(jnp.maximum(
                        ownR.astype(jnp.float32), 0.0)))
                    vL = (commL[recv_slot, sl, :].astype(jnp.float32)
                          + pbL[sl, :].astype(jnp.float32))
                    ownL = vL.astype(jnp.bfloat16)
                    commL[0, sl, :] = ownL
                    return jnp.maximum(mx, jnp.max(jnp.maximum(
                        ownL.astype(jnp.float32), 0.0)))
                own_max = lax.fori_loop(0, CH // TR, final_tile, own_max)

        amax_buf[d, :, :] = jnp.full((8, 128), own_max, dtype=jnp.float32)
        bcasts = []
        for k in range(1, N_DEV):
            j = lax.rem(d + k, N_DEV)
            bc = pltpu.make_async_remote_copy(
                src_ref=amax_buf.at[d],
                dst_ref=amax_buf.at[d],
                send_sem=bc_send_sems.at[k - 1],
                recv_sem=bc_recv_sems.at[k - 1],
                device_id=(j,),
                device_id_type=_MESH,
            )
            bc.start()
            bcasts.append(bc)
        for bc in bcasts:
            bc.wait_recv()
        for bc in bcasts:
            bc.wait_send()
        gmax = jnp.max(amax_buf[...])

        scale = gmax / 127.0
        inv = jnp.where(gmax > 0.0, 127.0 / gmax, 0.0)

        def quant_tile(t, _):
            sl = pl.ds(t * TR, TR)
            vR = commR[0, sl, :].astype(jnp.float32)
            qR = jnp.clip(jnp.round(jnp.maximum(vR, 0.0) * inv), 0.0, 127.0)
            qcommR[0, sl, :] = qR.astype(jnp.int8)
            commR[0, sl, :] = (qR * scale).astype(jnp.bfloat16)
            vL = commL[0, sl, :].astype(jnp.float32)
            qL = jnp.clip(jnp.round(jnp.maximum(vL, 0.0) * inv), 0.0, 127.0)
            qcommL[0, sl, :] = qL.astype(jnp.int8)
            commL[0, sl, :] = (qL * scale).astype(jnp.bfloat16)
            return 0
        lax.fori_loop(0, CH // TR, quant_tile, 0)

        odR = pltpu.make_async_copy(
            commR.at[0], out_ref.at[pl.ds(d * CH, CH), pl.ds(0, HN)], ob_semR)
        odR.start()
        odL = pltpu.make_async_copy(
            commL.at[0], out_ref.at[pl.ds(d * CH, CH), pl.ds(HN, HN)], ob_semL)
        odL.start()
        odR.wait()
        odL.wait()

        pl.semaphore_signal(ag_creditR, inc=1, device_id=(left,),
                            device_id_type=_MESH)
        pl.semaphore_signal(ag_creditL, inc=1, device_id=(right,),
                            device_id_type=_MESH)

        for s in range(N_DEV - 1):
            send_slot = s % 2
            recv_slot = (s + 1) % 2
            pl.semaphore_wait(ag_creditR, 1)
            pl.semaphore_wait(ag_creditL, 1)
            rdmaR = pltpu.make_async_remote_copy(
                src_ref=qcommR.at[send_slot], dst_ref=qcommR.at[recv_slot],
                send_sem=agR_send.at[s], recv_sem=agR_recv.at[s],
                device_id=(right,), device_id_type=_MESH)
            rdmaR.start()
            rdmaL = pltpu.make_async_remote_copy(
                src_ref=qcommL.at[send_slot], dst_ref=qcommL.at[recv_slot],
                send_sem=agL_send.at[s], recv_sem=agL_recv.at[s],
                device_id=(left,), device_id_type=_MESH)
            rdmaL.start()
            rdmaR.wait()
            rdmaL.wait()
            if s <= N_DEV - 3:
                pl.semaphore_signal(ag_creditR, inc=1, device_id=(left,),
                                    device_id_type=_MESH)
                pl.semaphore_signal(ag_creditL, inc=1, device_id=(right,),
                                    device_id_type=_MESH)

            def dequant_tile(t, _):
                sl = pl.ds(t * TR, TR)
                commR[recv_slot, sl, :] = (
                    qcommR[recv_slot, sl, :].astype(jnp.float32) * scale
                ).astype(jnp.bfloat16)
                commL[recv_slot, sl, :] = (
                    qcommL[recv_slot, sl, :].astype(jnp.float32) * scale
                ).astype(jnp.bfloat16)
                return 0
            lax.fori_loop(0, CH // TR, dequant_tile, 0)

            cR = lax.rem(d + (2 * N_DEV - 1 - s), N_DEV)
            cL = lax.rem(d + 1 + s, N_DEV)
            stR = pltpu.make_async_copy(
                commR.at[recv_slot],
                out_ref.at[pl.ds(cR * CH, CH), pl.ds(0, HN)], ob_semR)
            stR.start()
            stL = pltpu.make_async_copy(
                commL.at[recv_slot],
                out_ref.at[pl.ds(cL * CH, CH), pl.ds(HN, HN)], ob_semL)
            stL.start()
            stR.wait()
            stL.wait()

    return pl.pallas_call(
        body,
        out_shape=jax.ShapeDtypeStruct((M, N), jnp.bfloat16),
        in_specs=[pl.BlockSpec(memory_space=pl.ANY)],
        out_specs=pl.BlockSpec(memory_space=pl.ANY),
        scratch_shapes=[
            pltpu.VMEM((2, CH, HN), jnp.bfloat16),
            pltpu.VMEM((2, CH, HN), jnp.bfloat16),
            pltpu.VMEM((2, CH, HN), jnp.int8),
            pltpu.VMEM((2, CH, HN), jnp.int8),
            pltpu.VMEM((CH, HN), jnp.bfloat16),
            pltpu.VMEM((CH, HN), jnp.bfloat16),
            pltpu.VMEM((N_DEV, 8, 128), jnp.float32),
            pltpu.SemaphoreType.DMA,
            pltpu.SemaphoreType.DMA,
            pltpu.SemaphoreType.DMA,
            pltpu.SemaphoreType.DMA,
            pltpu.SemaphoreType.DMA((N_DEV - 1,)),
            pltpu.SemaphoreType.DMA((N_DEV - 1,)),
            pltpu.SemaphoreType.DMA((N_DEV - 1,)),
            pltpu.SemaphoreType.DMA((N_DEV - 1,)),
            pltpu.SemaphoreType.DMA((N_DEV - 1,)),
            pltpu.SemaphoreType.DMA((N_DEV - 1,)),
            pltpu.SemaphoreType.DMA((N_DEV - 1,)),
            pltpu.SemaphoreType.DMA((N_DEV - 1,)),
            pltpu.SemaphoreType.DMA((N_DEV - 1,)),
            pltpu.SemaphoreType.DMA((N_DEV - 1,)),
            pltpu.SemaphoreType.REGULAR,
            pltpu.SemaphoreType.REGULAR,
            pltpu.SemaphoreType.REGULAR,
            pltpu.SemaphoreType.REGULAR,
        ],
        compiler_params=pltpu.CompilerParams(
            collective_id=0,
            vmem_limit_bytes=48 * 1024 * 1024,
        ),
    )(partial)


def kernel(x, w_mat):
    partial = jnp.dot(
        x.astype(jnp.bfloat16), w_mat.astype(jnp.bfloat16),
        preferred_element_type=jnp.float32,
    ).astype(jnp.bfloat16)
    return _ar_epilogue(partial)


# device time: 640751 ns/iter; 2.2575x vs baseline; 1.0389x over previous
import jax
import jax.numpy as jnp
from jax import lax
from jax.experimental import pallas as pl
from jax.experimental.pallas import tpu as pltpu

N_DEV = 8
M, K, N = 4096, 4096, 8192
CH = M // N_DEV
HN = N // 2
TR = 128

_MESH = pl.DeviceIdType.MESH


def _fused_gemm_ar(x, w_mat):

    def body(x_ref, w_ref, out_ref, commR, commL, qcommR, qcommL,
             pbR, pbL, amax_buf,
             ob_semR, ob_semL,
             rsR_send, rsR_recv, rsL_send, rsL_recv,
             agR_send, agR_recv, agL_send, agL_recv,
             bc_send_sems, bc_recv_sems,
             rs_creditR, rs_creditL, ag_creditR, ag_creditL):
        d = lax.axis_index("i")
        left = lax.rem(d + (N_DEV - 1), N_DEV)
        right = lax.rem(d + 1, N_DEV)

        barrier = pltpu.get_barrier_semaphore()
        pl.semaphore_signal(barrier, inc=1, device_id=(left,),
                            device_id_type=_MESH)
        pl.semaphore_signal(barrier, inc=1, device_id=(right,),
                            device_id_type=_MESH)
        pl.semaphore_wait(barrier, 2)

        cR0 = lax.rem(d + (N_DEV - 1), N_DEV)
        cL0 = lax.rem(d + 1, N_DEV)

        def gemm_tiles(c_right, c_left, dstR, dstL):
            def tile(t, _):
                sl = pl.ds(t * TR, TR)
                xt = x_ref[pl.ds(c_right * CH + t * TR, TR), :]
                dstR[sl, :] = jnp.dot(
                    xt, w_ref[:, 0:HN],
                    preferred_element_type=jnp.float32,
                ).astype(jnp.bfloat16)
                xt2 = x_ref[pl.ds(c_left * CH + t * TR, TR), :]
                dstL[sl, :] = jnp.dot(
                    xt2, w_ref[:, HN:],
                    preferred_element_type=jnp.float32,
                ).astype(jnp.bfloat16)
                return 0
            lax.fori_loop(0, CH // TR, tile, 0)

        gemm_tiles(cR0, cL0, commR.at[0], commL.at[0])

        own_max = jnp.float32(0.0)
        for s in range(N_DEV - 1):
            send_slot = s % 2
            recv_slot = (s + 1) % 2
            if s >= 1:
                pl.semaphore_wait(rs_creditR, 1)
                pl.semaphore_wait(rs_creditL, 1)
            rdmaR = pltpu.make_async_remote_copy(
                src_ref=commR.at[send_slot], dst_ref=commR.at[recv_slot],
                send_sem=rsR_send.at[s], recv_sem=rsR_recv.at[s],
                device_id=(right,), device_id_type=_MESH)
            rdmaR.start()
            rdmaL = pltpu.make_async_remote_copy(
                src_ref=commL.at[send_slot], dst_ref=commL.at[recv_slot],
                send_sem=rsL_send.at[s], recv_sem=rsL_recv.at[s],
                device_id=(left,), device_id_type=_MESH)
            rdmaL.start()
            cR = lax.rem(d + (2 * N_DEV - 2 - s), N_DEV)
            cL = lax.rem(d + 2 + s, N_DEV)
            gemm_tiles(cR, cL, pbR, pbL)
            rdmaR.wait()
            rdmaL.wait()
            if s <= N_DEV - 3:
                pl.semaphore_signal(rs_creditR, inc=1, device_id=(left,),
                                    device_id_type=_MESH)
                pl.semaphore_signal(rs_creditL, inc=1, device_id=(right,),
                                    device_id_type=_MESH)

            if s < N_DEV - 2:
                def add_tile(t, _):
                    sl = pl.ds(t * TR, TR)
                    vR = (commR[recv_slot, sl, :].astype(jnp.float32)
                          + pbR[sl, :].astype(jnp.float32))
                    commR[recv_slot, sl, :] = vR.astype(jnp.bfloat16)
                    vL = (commL[recv_slot, sl, :].astype(jnp.float32)
                          + pbL[sl, :].astype(jnp.float32))
                    commL[recv_slot, sl, :] = vL.astype(jnp.bfloat16)
                    return 0
                lax.fori_loop(0, CH // TR, add_tile, 0)
            else:
                def final_tile(t, mx):
                    sl = pl.ds(t * TR, TR)
                    vR = (commR[recv_slot, sl, :].astype(jnp.float32)
                          + pbR[sl, :].astype(jnp.float32))
                    ownR = vR.astype(jnp.bfloat16)
                    commR[0, sl, :] = ownR
                    mx = jnp.maximum(mx, jnp.max(jnp.maximum(
                        ownR.astype(jnp.float32), 0.0)))
                    vL = (commL[recv_slot, sl, :].astype(jnp.float32)
                          + pbL[sl, :].astype(jnp.float32))
                    ownL = vL.astype(jnp.bfloat16)
                    commL[0, sl, :] = ownL
                    return jnp.maximum(mx, jnp.max(jnp.maximum(
                        ownL.astype(jnp.float32), 0.0)))
                own_max = lax.fori_loop(0, CH // TR, final_tile, own_max)

        amax_buf[d, :, :] = jnp.full((8, 128), own_max, dtype=jnp.float32)
        bcasts = []
        for k in range(1, N_DEV):
            j = lax.rem(d + k, N_DEV)
            bc = pltpu.make_async_remote_copy(
                src_ref=amax_buf.at[d],
                dst_ref=amax_buf.at[d],
                send_sem=bc_send_sems.at[k - 1],
                recv_sem=bc_recv_sems.at[k - 1],
                device_id=(j,),
                device_id_type=_MESH,
            )
            bc.start()
            bcasts.append(bc)
        for bc in bcasts:
            bc.wait_recv()
        for bc in bcasts:
            bc.wait_send()
        gmax = jnp.max(amax_buf[...])

        scale = gmax / 127.0
        inv = jnp.where(gmax > 0.0, 127.0 / gmax, 0.0)

        def quant_tile(t, _):
            sl = pl.ds(t * TR, TR)
            vR = commR[0, sl, :].astype(jnp.float32)
            qR = jnp.clip(jnp.round(jnp.maximum(vR, 0.0) * inv), 0.0, 127.0)
            qcommR[0, sl, :] = qR.astype(jnp.int8)
            commR[0, sl, :] = (qR * scale).astype(jnp.bfloat16)
            vL = commL[0, sl, :].astype(jnp.float32)
            qL = jnp.clip(jnp.round(jnp.maximum(vL, 0.0) * inv), 0.0, 127.0)
            qcommL[0, sl, :] = qL.astype(jnp.int8)
            commL[0, sl, :] = (qL * scale).astype(jnp.bfloat16)
            return 0
        lax.fori_loop(0, CH // TR, quant_tile, 0)

        odR = pltpu.make_async_copy(
            commR.at[0], out_ref.at[pl.ds(d * CH, CH), pl.ds(0, HN)], ob_semR)
        odR.start()
        odL = pltpu.make_async_copy(
            commL.at[0], out_ref.at[pl.ds(d * CH, CH), pl.ds(HN, HN)], ob_semL)
        odL.start()
        odR.wait()
        odL.wait()

        pl.semaphore_signal(ag_creditR, inc=1, device_id=(left,),
                            device_id_type=_MESH)
        pl.semaphore_signal(ag_creditL, inc=1, device_id=(right,),
                            device_id_type=_MESH)

        for s in range(N_DEV - 1):
            send_slot = s % 2
            recv_slot = (s + 1) % 2
            pl.semaphore_wait(ag_creditR, 1)
            pl.semaphore_wait(ag_creditL, 1)
            rdmaR = pltpu.make_async_remote_copy(
                src_ref=qcommR.at[send_slot], dst_ref=qcommR.at[recv_slot],
                send_sem=agR_send.at[s], recv_sem=agR_recv.at[s],
                device_id=(right,), device_id_type=_MESH)
            rdmaR.start()
            rdmaL = pltpu.make_async_remote_copy(
                src_ref=qcommL.at[send_slot], dst_ref=qcommL.at[recv_slot],
                send_sem=agL_send.at[s], recv_sem=agL_recv.at[s],
                device_id=(left,), device_id_type=_MESH)
            rdmaL.start()
            rdmaR.wait()
            rdmaL.wait()
            if s <= N_DEV - 3:
                pl.semaphore_signal(ag_creditR, inc=1, device_id=(left,),
                                    device_id_type=_MESH)
                pl.semaphore_signal(ag_creditL, inc=1, device_id=(right,),
                                    device_id_type=_MESH)

            def dequant_tile(t, _):
                sl = pl.ds(t * TR, TR)
                commR[recv_slot, sl, :] = (
                    qcommR[recv_slot, sl, :].astype(jnp.float32) * scale
                ).astype(jnp.bfloat16)
                commL[recv_slot, sl, :] = (
                    qcommL[recv_slot, sl, :].astype(jnp.float32) * scale
                ).astype(jnp.bfloat16)
                return 0
            lax.fori_loop(0, CH // TR, dequant_tile, 0)

            cR = lax.rem(d + (2 * N_DEV - 1 - s), N_DEV)
            cL = lax.rem(d + 1 + s, N_DEV)
            stR = pltpu.make_async_copy(
                commR.at[recv_slot],
                out_ref.at[pl.ds(cR * CH, CH), pl.ds(0, HN)], ob_semR)
            stR.start()
            stL = pltpu.make_async_copy(
                commL.at[recv_slot],
                out_ref.at[pl.ds(cL * CH, CH), pl.ds(HN, HN)], ob_semL)
            stL.start()
            stR.wait()
            stL.wait()

    return pl.pallas_call(
        body,
        out_shape=jax.ShapeDtypeStruct((M, N), jnp.bfloat16),
        in_specs=[pl.BlockSpec(memory_space=pltpu.VMEM),
                  pl.BlockSpec(memory_space=pltpu.VMEM)],
        out_specs=pl.BlockSpec(memory_space=pl.ANY),
        scratch_shapes=[
            pltpu.VMEM((2, CH, HN), jnp.bfloat16),
            pltpu.VMEM((2, CH, HN), jnp.bfloat16),
            pltpu.VMEM((2, CH, HN), jnp.int8),
            pltpu.VMEM((2, CH, HN), jnp.int8),
            pltpu.VMEM((CH, HN), jnp.bfloat16),
            pltpu.VMEM((CH, HN), jnp.bfloat16),
            pltpu.VMEM((N_DEV, 8, 128), jnp.float32),
            pltpu.SemaphoreType.DMA,
            pltpu.SemaphoreType.DMA,
            pltpu.SemaphoreType.DMA((N_DEV - 1,)),
            pltpu.SemaphoreType.DMA((N_DEV - 1,)),
            pltpu.SemaphoreType.DMA((N_DEV - 1,)),
            pltpu.SemaphoreType.DMA((N_DEV - 1,)),
            pltpu.SemaphoreType.DMA((N_DEV - 1,)),
            pltpu.SemaphoreType.DMA((N_DEV - 1,)),
            pltpu.SemaphoreType.DMA((N_DEV - 1,)),
            pltpu.SemaphoreType.DMA((N_DEV - 1,)),
            pltpu.SemaphoreType.DMA((N_DEV - 1,)),
            pltpu.SemaphoreType.DMA((N_DEV - 1,)),
            pltpu.SemaphoreType.REGULAR,
            pltpu.SemaphoreType.REGULAR,
            pltpu.SemaphoreType.REGULAR,
            pltpu.SemaphoreType.REGULAR,
        ],
        compiler_params=pltpu.CompilerParams(
            collective_id=0,
            vmem_limit_bytes=58 * 1024 * 1024,
        ),
    )(x, w_mat)


def kernel(x, w_mat):
    return _fused_gemm_ar(
        x.astype(jnp.bfloat16), w_mat.astype(jnp.bfloat16))


# device time: 616658 ns/iter; 2.3457x vs baseline; 1.0391x over previous
import jax
import jax.numpy as jnp
from jax import lax
from jax.experimental import pallas as pl
from jax.experimental.pallas import tpu as pltpu

N_DEV = 8
M, K, N = 4096, 4096, 8192
CH = M // N_DEV
HN = N // 2
TR = 128

_MESH = pl.DeviceIdType.MESH


def _fused_gemm_ar(x, w_mat):

    def body(x_ref, w_ref, out_ref, commR, commL, qcommR, qcommL,
             pbR, pbL, amax_buf,
             ob_semR, ob_semL,
             rsR_send, rsR_recv, rsL_send, rsL_recv,
             agR_send, agR_recv, agL_send, agL_recv,
             bc_send_sems, bc_recv_sems,
             rs_creditR, rs_creditL, ag_creditR, ag_creditL):
        d = lax.axis_index("i")
        left = lax.rem(d + (N_DEV - 1), N_DEV)
        right = lax.rem(d + 1, N_DEV)

        barrier = pltpu.get_barrier_semaphore()
        pl.semaphore_signal(barrier, inc=1, device_id=(left,),
                            device_id_type=_MESH)
        pl.semaphore_signal(barrier, inc=1, device_id=(right,),
                            device_id_type=_MESH)
        pl.semaphore_wait(barrier, 2)

        cR0 = lax.rem(d + (N_DEV - 1), N_DEV)
        cL0 = lax.rem(d + 1, N_DEV)

        def gemm_tiles(c_right, c_left, dstR, dstL):
            def tile(t, _):
                sl = pl.ds(t * TR, TR)
                xt = x_ref[pl.ds(c_right * CH + t * TR, TR), :]
                dstR[sl, :] = jnp.dot(
                    xt, w_ref[:, 0:HN],
                    preferred_element_type=jnp.float32,
                ).astype(jnp.bfloat16)
                xt2 = x_ref[pl.ds(c_left * CH + t * TR, TR), :]
                dstL[sl, :] = jnp.dot(
                    xt2, w_ref[:, HN:],
                    preferred_element_type=jnp.float32,
                ).astype(jnp.bfloat16)
                return 0
            lax.fori_loop(0, CH // TR, tile, 0)

        gemm_tiles(cR0, cL0, commR.at[0], commL.at[0])

        own_max = jnp.float32(0.0)
        for s in range(N_DEV - 1):
            send_slot = s % 2
            recv_slot = (s + 1) % 2
            if s >= 1:
                pl.semaphore_wait(rs_creditR, 1)
                pl.semaphore_wait(rs_creditL, 1)
            rdmaR = pltpu.make_async_remote_copy(
                src_ref=commR.at[send_slot], dst_ref=commR.at[recv_slot],
                send_sem=rsR_send.at[s], recv_sem=rsR_recv.at[s],
                device_id=(right,), device_id_type=_MESH)
            rdmaR.start()
            rdmaL = pltpu.make_async_remote_copy(
                src_ref=commL.at[send_slot], dst_ref=commL.at[recv_slot],
                send_sem=rsL_send.at[s], recv_sem=rsL_recv.at[s],
                device_id=(left,), device_id_type=_MESH)
            rdmaL.start()
            cR = lax.rem(d + (2 * N_DEV - 2 - s), N_DEV)
            cL = lax.rem(d + 2 + s, N_DEV)
            gemm_tiles(cR, cL, pbR, pbL)
            rdmaR.wait()
            rdmaL.wait()
            if s <= N_DEV - 3:
                pl.semaphore_signal(rs_creditR, inc=1, device_id=(left,),
                                    device_id_type=_MESH)
                pl.semaphore_signal(rs_creditL, inc=1, device_id=(right,),
                                    device_id_type=_MESH)

            if s < N_DEV - 2:
                def add_tile(t, _):
                    sl = pl.ds(t * TR, TR)
                    vR = (commR[recv_slot, sl, :].astype(jnp.float32)
                          + pbR[sl, :].astype(jnp.float32))
                    commR[recv_slot, sl, :] = vR.astype(jnp.bfloat16)
                    vL = (commL[recv_slot, sl, :].astype(jnp.float32)
                          + pbL[sl, :].astype(jnp.float32))
                    commL[recv_slot, sl, :] = vL.astype(jnp.bfloat16)
                    return 0
                lax.fori_loop(0, CH // TR, add_tile, 0)
            else:
                def final_tile(t, mx):
                    sl = pl.ds(t * TR, TR)
                    vR = (commR[recv_slot, sl, :].astype(jnp.float32)
                          + pbR[sl, :].astype(jnp.float32))
                    ownR = vR.astype(jnp.bfloat16)
                    commR[0, sl, :] = ownR
                    mx = jnp.maximum(mx, jnp.max(jnp.maximum(
                        ownR.astype(jnp.float32), 0.0)))
                    vL = (commL[recv_slot, sl, :].astype(jnp.float32)
                          + pbL[sl, :].astype(jnp.float32))
                    ownL = vL.astype(jnp.bfloat16)
                    commL[0, sl, :] = ownL
                    return jnp.maximum(mx, jnp.max(jnp.maximum(
                        ownL.astype(jnp.float32), 0.0)))
                own_max = lax.fori_loop(0, CH // TR, final_tile, own_max)

        amax_buf[d, :, :] = jnp.full((8, 128), own_max, dtype=jnp.float32)
        bcasts = []
        for k in range(1, N_DEV):
            j = lax.rem(d + k, N_DEV)
            bc = pltpu.make_async_remote_copy(
                src_ref=amax_buf.at[d],
                dst_ref=amax_buf.at[d],
                send_sem=bc_send_sems.at[k - 1],
                recv_sem=bc_recv_sems.at[k - 1],
                device_id=(j,),
                device_id_type=_MESH,
            )
            bc.start()
            bcasts.append(bc)
        for bc in bcasts:
            bc.wait_recv()
        for bc in bcasts:
            bc.wait_send()
        gmax = jnp.max(amax_buf[...])

        scale = gmax / 127.0
        inv = jnp.where(gmax > 0.0, 127.0 / gmax, 0.0)

        def quant_tile(t, _):
            sl = pl.ds(t * TR, TR)
            vR = commR[0, sl, :].astype(jnp.float32)
            qR = jnp.clip(jnp.round(jnp.maximum(vR, 0.0) * inv), 0.0, 127.0)
            qcommR[0, sl, :] = qR.astype(jnp.int8)
            commR[0, sl, :] = (qR * scale).astype(jnp.bfloat16)
            vL = commL[0, sl, :].astype(jnp.float32)
            qL = jnp.clip(jnp.round(jnp.maximum(vL, 0.0) * inv), 0.0, 127.0)
            qcommL[0, sl, :] = qL.astype(jnp.int8)
            commL[0, sl, :] = (qL * scale).astype(jnp.bfloat16)
            return 0
        lax.fori_loop(0, CH // TR, quant_tile, 0)

        odR = pltpu.make_async_copy(
            commR.at[0], out_ref.at[pl.ds(d * CH, CH), pl.ds(0, HN)], ob_semR)
        odR.start()
        odL = pltpu.make_async_copy(
            commL.at[0], out_ref.at[pl.ds(d * CH, CH), pl.ds(HN, HN)], ob_semL)
        odL.start()
        odR.wait()
        odL.wait()

        pl.semaphore_signal(ag_creditR, inc=1, device_id=(left,),
                            device_id_type=_MESH)
        pl.semaphore_signal(ag_creditL, inc=1, device_id=(right,),
                            device_id_type=_MESH)

        def ag_rdma(s):
            rR = pltpu.make_async_remote_copy(
                src_ref=qcommR.at[s % 2], dst_ref=qcommR.at[(s + 1) % 2],
                send_sem=agR_send.at[s], recv_sem=agR_recv.at[s],
                device_id=(right,), device_id_type=_MESH)
            rL = pltpu.make_async_remote_copy(
                src_ref=qcommL.at[s % 2], dst_ref=qcommL.at[(s + 1) % 2],
                send_sem=agL_send.at[s], recv_sem=agL_recv.at[s],
                device_id=(left,), device_id_type=_MESH)
            rR.start()
            rL.start()
            return rR, rL

        pl.semaphore_wait(ag_creditR, 1)
        pl.semaphore_wait(ag_creditL, 1)
        cur = ag_rdma(0)
        for s in range(N_DEV - 1):
            recv_slot = (s + 1) % 2
            cur[0].wait()
            cur[1].wait()
            if s <= N_DEV - 3:
                pl.semaphore_signal(ag_creditR, inc=1, device_id=(left,),
                                    device_id_type=_MESH)
                pl.semaphore_signal(ag_creditL, inc=1, device_id=(right,),
                                    device_id_type=_MESH)
                pl.semaphore_wait(ag_creditR, 1)
                pl.semaphore_wait(ag_creditL, 1)
                cur = ag_rdma(s + 1)

            def dequant_tile(t, _):
                sl = pl.ds(t * TR, TR)
                commR[recv_slot, sl, :] = (
                    qcommR[recv_slot, sl, :].astype(jnp.float32) * scale
                ).astype(jnp.bfloat16)
                commL[recv_slot, sl, :] = (
                    qcommL[recv_slot, sl, :].astype(jnp.float32) * scale
                ).astype(jnp.bfloat16)
                return 0
            lax.fori_loop(0, CH // TR, dequant_tile, 0)

            cR = lax.rem(d + (2 * N_DEV - 1 - s), N_DEV)
            cL = lax.rem(d + 1 + s, N_DEV)
            stR = pltpu.make_async_copy(
                commR.at[recv_slot],
                out_ref.at[pl.ds(cR * CH, CH), pl.ds(0, HN)], ob_semR)
            stR.start()
            stL = pltpu.make_async_copy(
                commL.at[recv_slot],
                out_ref.at[pl.ds(cL * CH, CH), pl.ds(HN, HN)], ob_semL)
            stL.start()
            stR.wait()
            stL.wait()

    return pl.pallas_call(
        body,
        out_shape=jax.ShapeDtypeStruct((M, N), jnp.bfloat16),
        in_specs=[pl.BlockSpec(memory_space=pltpu.VMEM),
                  pl.BlockSpec(memory_space=pltpu.VMEM)],
        out_specs=pl.BlockSpec(memory_space=pl.ANY),
        scratch_shapes=[
            pltpu.VMEM((2, CH, HN), jnp.bfloat16),
            pltpu.VMEM((2, CH, HN), jnp.bfloat16),
            pltpu.VMEM((2, CH, HN), jnp.int8),
            pltpu.VMEM((2, CH, HN), jnp.int8),
            pltpu.VMEM((CH, HN), jnp.bfloat16),
            pltpu.VMEM((CH, HN), jnp.bfloat16),
            pltpu.VMEM((N_DEV, 8, 128), jnp.float32),
            pltpu.SemaphoreType.DMA,
            pltpu.SemaphoreType.DMA,
            pltpu.SemaphoreType.DMA((N_DEV - 1,)),
            pltpu.SemaphoreType.DMA((N_DEV - 1,)),
            pltpu.SemaphoreType.DMA((N_DEV - 1,)),
            pltpu.SemaphoreType.DMA((N_DEV - 1,)),
            pltpu.SemaphoreType.DMA((N_DEV - 1,)),
            pltpu.SemaphoreType.DMA((N_DEV - 1,)),
            pltpu.SemaphoreType.DMA((N_DEV - 1,)),
            pltpu.SemaphoreType.DMA((N_DEV - 1,)),
            pltpu.SemaphoreType.DMA((N_DEV - 1,)),
            pltpu.SemaphoreType.DMA((N_DEV - 1,)),
            pltpu.SemaphoreType.REGULAR,
            pltpu.SemaphoreType.REGULAR,
            pltpu.SemaphoreType.REGULAR,
            pltpu.SemaphoreType.REGULAR,
        ],
        compiler_params=pltpu.CompilerParams(
            collective_id=0,
            vmem_limit_bytes=58 * 1024 * 1024,
        ),
    )(x, w_mat)


def kernel(x, w_mat):
    return _fused_gemm_ar(
        x.astype(jnp.bfloat16), w_mat.astype(jnp.bfloat16))


# device time: 616456 ns/iter; 2.3464x vs baseline; 1.0003x over previous
import jax
import jax.numpy as jnp
from jax import lax
from jax.experimental import pallas as pl
from jax.experimental.pallas import tpu as pltpu

N_DEV = 8
M, K, N = 4096, 4096, 8192
CH = M // N_DEV
HN = N // 2
TR = 128

_MESH = pl.DeviceIdType.MESH


def _fused_gemm_ar(x, w_mat):

    def body(x_ref, w_ref, out_ref, commR, commL, qcommR, qcommL,
             pbR, pbL, amax_buf,
             ob_semR, ob_semL,
             rsR_send, rsR_recv, rsL_send, rsL_recv,
             agR_send, agR_recv, agL_send, agL_recv,
             bc_send_sems, bc_recv_sems,
             rs_creditR, rs_creditL, ag_creditR, ag_creditL):
        d = lax.axis_index("i")
        left = lax.rem(d + (N_DEV - 1), N_DEV)
        right = lax.rem(d + 1, N_DEV)

        cR0 = lax.rem(d + (N_DEV - 1), N_DEV)
        cL0 = lax.rem(d + 1, N_DEV)

        def gemm_tiles(c_right, c_left, dstR, dstL):
            def tile(t, _):
                sl = pl.ds(t * TR, TR)
                xt = x_ref[pl.ds(c_right * CH + t * TR, TR), :]
                dstR[sl, :] = jnp.dot(
                    xt, w_ref[:, 0:HN],
                    preferred_element_type=jnp.float32,
                ).astype(jnp.bfloat16)
                xt2 = x_ref[pl.ds(c_left * CH + t * TR, TR), :]
                dstL[sl, :] = jnp.dot(
                    xt2, w_ref[:, HN:],
                    preferred_element_type=jnp.float32,
                ).astype(jnp.bfloat16)
                return 0
            lax.fori_loop(0, CH // TR, tile, 0)

        gemm_tiles(cR0, cL0, commR.at[0], commL.at[0])

        barrier = pltpu.get_barrier_semaphore()
        pl.semaphore_signal(barrier, inc=1, device_id=(left,),
                            device_id_type=_MESH)
        pl.semaphore_signal(barrier, inc=1, device_id=(right,),
                            device_id_type=_MESH)
        pl.semaphore_wait(barrier, 2)

        own_max = jnp.float32(0.0)
        for s in range(N_DEV - 1):
            send_slot = s % 2
            recv_slot = (s + 1) % 2
            if s >= 1:
                pl.semaphore_wait(rs_creditR, 1)
                pl.semaphore_wait(rs_creditL, 1)
            rdmaR = pltpu.make_async_remote_copy(
                src_ref=commR.at[send_slot], dst_ref=commR.at[recv_slot],
                send_sem=rsR_send.at[s], recv_sem=rsR_recv.at[s],
                device_id=(right,), device_id_type=_MESH)
            rdmaR.start()
            rdmaL = pltpu.make_async_remote_copy(
                src_ref=commL.at[send_slot], dst_ref=commL.at[recv_slot],
                send_sem=rsL_send.at[s], recv_sem=rsL_recv.at[s],
                device_id=(left,), device_id_type=_MESH)
            rdmaL.start()
            cR = lax.rem(d + (2 * N_DEV - 2 - s), N_DEV)
            cL = lax.rem(d + 2 + s, N_DEV)
            gemm_tiles(cR, cL, pbR, pbL)
            rdmaR.wait()
            rdmaL.wait()
            if s <= N_DEV - 3:
                pl.semaphore_signal(rs_creditR, inc=1, device_id=(left,),
                                    device_id_type=_MESH)
                pl.semaphore_signal(rs_creditL, inc=1, device_id=(right,),
                                    device_id_type=_MESH)

            if s < N_DEV - 2:
                def add_tile(t, _):
                    sl = pl.ds(t * TR, TR)
                    vR = (commR[recv_slot, sl, :].astype(jnp.float32)
                          + pbR[sl, :].astype(jnp.float32))
                    commR[recv_slot, sl, :] = vR.astype(jnp.bfloat16)
                    vL = (commL[recv_slot, sl, :].astype(jnp.float32)
                          + pbL[sl, :].astype(jnp.float32))
                    commL[recv_slot, sl, :] = vL.astype(jnp.bfloat16)
                    return 0
                lax.fori_loop(0, CH // TR, add_tile, 0)
            else:
                def final_tile(t, mx):
                    sl = pl.ds(t * TR, TR)
                    vR = (commR[recv_slot, sl, :].astype(jnp.float32)
                          + pbR[sl, :].astype(jnp.float32))
                    ownR = vR.astype(jnp.bfloat16)
                    commR[0, sl, :] = ownR
                    mx = jnp.maximum(mx, jnp.max(jnp.maximum(
                        ownR.astype(jnp.float32), 0.0)))
                    vL = (commL[recv_slot, sl, :].astype(jnp.float32)
                          + pbL[sl, :].astype(jnp.float32))
                    ownL = vL.astype(jnp.bfloat16)
                    commL[0, sl, :] = ownL
                    return jnp.maximum(mx, jnp.max(jnp.maximum(
                        ownL.astype(jnp.float32), 0.0)))
                own_max = lax.fori_loop(0, CH // TR, final_tile, own_max)

        amax_buf[d, :, :] = jnp.full((8, 128), own_max, dtype=jnp.float32)
        bcasts = []
        for k in range(1, N_DEV):
            j = lax.rem(d + k, N_DEV)
            bc = pltpu.make_async_remote_copy(
                src_ref=amax_buf.at[d],
                dst_ref=amax_buf.at[d],
                send_sem=bc_send_sems.at[k - 1],
                recv_sem=bc_recv_sems.at[k - 1],
                device_id=(j,),
                device_id_type=_MESH,
            )
            bc.start()
            bcasts.append(bc)
        for bc in bcasts:
            bc.wait_recv()
        for bc in bcasts:
            bc.wait_send()
        gmax = jnp.max(amax_buf[...])

        scale = gmax / 127.0
        inv = jnp.where(gmax > 0.0, 127.0 / gmax, 0.0)

        def quant_tile(t, _):
            sl = pl.ds(t * TR, TR)
            vR = commR[0, sl, :].astype(jnp.float32)
            qR = jnp.clip(jnp.round(jnp.maximum(vR, 0.0) * inv), 0.0, 127.0)
            qcommR[0, sl, :] = qR.astype(jnp.int8)
            commR[0, sl, :] = (qR * scale).astype(jnp.bfloat16)
            vL = commL[0, sl, :].astype(jnp.float32)
            qL = jnp.clip(jnp.round(jnp.maximum(vL, 0.0) * inv), 0.0, 127.0)
            qcommL[0, sl, :] = qL.astype(jnp.int8)
            commL[0, sl, :] = (qL * scale).astype(jnp.bfloat16)
            return 0
        lax.fori_loop(0, CH // TR, quant_tile, 0)

        odR = pltpu.make_async_copy(
            commR.at[0], out_ref.at[pl.ds(d * CH, CH), pl.ds(0, HN)], ob_semR)
        odR.start()
        odL = pltpu.make_async_copy(
            commL.at[0], out_ref.at[pl.ds(d * CH, CH), pl.ds(HN, HN)], ob_semL)
        odL.start()
        odR.wait()
        odL.wait()

        pl.semaphore_signal(ag_creditR, inc=1, device_id=(left,),
                            device_id_type=_MESH)
        pl.semaphore_signal(ag_creditL, inc=1, device_id=(right,),
                            device_id_type=_MESH)

        def ag_rdma(s):
            rR = pltpu.make_async_remote_copy(
                src_ref=qcommR.at[s % 2], dst_ref=qcommR.at[(s + 1) % 2],
                send_sem=agR_send.at[s], recv_sem=agR_recv.at[s],
                device_id=(right,), device_id_type=_MESH)
            rL = pltpu.make_async_remote_copy(
                src_ref=qcommL.at[s % 2], dst_ref=qcommL.at[(s + 1) % 2],
                send_sem=agL_send.at[s], recv_sem=agL_recv.at[s],
                device_id=(left,), device_id_type=_MESH)
            rR.start()
            rL.start()
            return rR, rL

        pl.semaphore_wait(ag_creditR, 1)
        pl.semaphore_wait(ag_creditL, 1)
        cur = ag_rdma(0)
        for s in range(N_DEV - 1):
            recv_slot = (s + 1) % 2
            cur[0].wait()
            cur[1].wait()
            if s <= N_DEV - 3:
                pl.semaphore_signal(ag_creditR, inc=1, device_id=(left,),
                                    device_id_type=_MESH)
                pl.semaphore_signal(ag_creditL, inc=1, device_id=(right,),
                                    device_id_type=_MESH)
                pl.semaphore_wait(ag_creditR, 1)
                pl.semaphore_wait(ag_creditL, 1)
                cur = ag_rdma(s + 1)

            def dequant_tile(t, _):
                sl = pl.ds(t * TR, TR)
                commR[recv_slot, sl, :] = (
                    qcommR[recv_slot, sl, :].astype(jnp.float32) * scale
                ).astype(jnp.bfloat16)
                commL[recv_slot, sl, :] = (
                    qcommL[recv_slot, sl, :].astype(jnp.float32) * scale
                ).astype(jnp.bfloat16)
                return 0
            lax.fori_loop(0, CH // TR, dequant_tile, 0)

            cR = lax.rem(d + (2 * N_DEV - 1 - s), N_DEV)
            cL = lax.rem(d + 1 + s, N_DEV)
            stR = pltpu.make_async_copy(
                commR.at[recv_slot],
                out_ref.at[pl.ds(cR * CH, CH), pl.ds(0, HN)], ob_semR)
            stR.start()
            stL = pltpu.make_async_copy(
                commL.at[recv_slot],
                out_ref.at[pl.ds(cL * CH, CH), pl.ds(HN, HN)], ob_semL)
            stL.start()
            stR.wait()
            stL.wait()

    return pl.pallas_call(
        body,
        out_shape=jax.ShapeDtypeStruct((M, N), jnp.bfloat16),
        in_specs=[pl.BlockSpec(memory_space=pltpu.VMEM),
                  pl.BlockSpec(memory_space=pltpu.VMEM)],
        out_specs=pl.BlockSpec(memory_space=pl.ANY),
        scratch_shapes=[
            pltpu.VMEM((2, CH, HN), jnp.bfloat16),
            pltpu.VMEM((2, CH, HN), jnp.bfloat16),
            pltpu.VMEM((2, CH, HN), jnp.int8),
            pltpu.VMEM((2, CH, HN), jnp.int8),
            pltpu.VMEM((CH, HN), jnp.bfloat16),
            pltpu.VMEM((CH, HN), jnp.bfloat16),
            pltpu.VMEM((N_DEV, 8, 128), jnp.float32),
            pltpu.SemaphoreType.DMA,
            pltpu.SemaphoreType.DMA,
            pltpu.SemaphoreType.DMA((N_DEV - 1,)),
            pltpu.SemaphoreType.DMA((N_DEV - 1,)),
            pltpu.SemaphoreType.DMA((N_DEV - 1,)),
            pltpu.SemaphoreType.DMA((N_DEV - 1,)),
            pltpu.SemaphoreType.DMA((N_DEV - 1,)),
            pltpu.SemaphoreType.DMA((N_DEV - 1,)),
            pltpu.SemaphoreType.DMA((N_DEV - 1,)),
            pltpu.SemaphoreType.DMA((N_DEV - 1,)),
            pltpu.SemaphoreType.DMA((N_DEV - 1,)),
            pltpu.SemaphoreType.DMA((N_DEV - 1,)),
            pltpu.SemaphoreType.REGULAR,
            pltpu.SemaphoreType.REGULAR,
            pltpu.SemaphoreType.REGULAR,
            pltpu.SemaphoreType.REGULAR,
        ],
        compiler_params=pltpu.CompilerParams(
            collective_id=0,
            vmem_limit_bytes=58 * 1024 * 1024,
        ),
    )(x, w_mat)


def kernel(x, w_mat):
    return _fused_gemm_ar(
        x.astype(jnp.bfloat16), w_mat.astype(jnp.bfloat16))


# device time: 588974 ns/iter; 2.4559x vs baseline; 1.0467x over previous
import jax
import jax.numpy as jnp
from jax import lax
from jax.experimental import pallas as pl
from jax.experimental.pallas import tpu as pltpu

N_DEV = 8
M, K, N = 4096, 4096, 8192
CH = M // N_DEV
HN = N // 2
QN = HN // 2
TR = 128

_MESH = pl.DeviceIdType.MESH


def _fused_gemm_ar(x, w_mat):

    def body(x_ref, w_ref, out_ref, commR, commL, qcommR, qcommL,
             pbR, pbL, amax_buf,
             ob_semR, ob_semL,
             rsR_send, rsR_recv, rsL_send, rsL_recv,
             agR_send, agR_recv, agL_send, agL_recv,
             bc_send_sems, bc_recv_sems,
             rs_creditR, rs_creditL, ag_creditR, ag_creditL):
        d = lax.axis_index("i")
        left = lax.rem(d + (N_DEV - 1), N_DEV)
        right = lax.rem(d + 1, N_DEV)

        cR0 = lax.rem(d + (N_DEV - 1), N_DEV)
        cL0 = lax.rem(d + 1, N_DEV)

        def gemm_tiles(c_right, c_left, dstR, dstL):
            def tile(t, _):
                sl = pl.ds(t * TR, TR)
                xt = x_ref[pl.ds(c_right * CH + t * TR, TR), :]
                dstR[sl, :] = jnp.dot(
                    xt, w_ref[:, 0:HN],
                    preferred_element_type=jnp.float32,
                ).astype(jnp.bfloat16)
                xt2 = x_ref[pl.ds(c_left * CH + t * TR, TR), :]
                dstL[sl, :] = jnp.dot(
                    xt2, w_ref[:, HN:],
                    preferred_element_type=jnp.float32,
                ).astype(jnp.bfloat16)
                return 0
            lax.fori_loop(0, CH // TR, tile, 0)

        gemm_tiles(cR0, cL0, commR.at[0], commL.at[0])

        barrier = pltpu.get_barrier_semaphore()
        pl.semaphore_signal(barrier, inc=1, device_id=(left,),
                            device_id_type=_MESH)
        pl.semaphore_signal(barrier, inc=1, device_id=(right,),
                            device_id_type=_MESH)
        pl.semaphore_wait(barrier, 2)

        def rs_rdma(s, q):
            cs = pl.ds(q * QN, QN)
            rR = pltpu.make_async_remote_copy(
                src_ref=commR.at[s % 2, :, cs],
                dst_ref=commR.at[(s + 1) % 2, :, cs],
                send_sem=rsR_send.at[s, q], recv_sem=rsR_recv.at[s, q],
                device_id=(right,), device_id_type=_MESH)
            rL = pltpu.make_async_remote_copy(
                src_ref=commL.at[s % 2, :, cs],
                dst_ref=commL.at[(s + 1) % 2, :, cs],
                send_sem=rsL_send.at[s, q], recv_sem=rsL_recv.at[s, q],
                device_id=(left,), device_id_type=_MESH)
            rR.start()
            rL.start()
            return rR, rL

        def rs_gemm(s):
            cR = lax.rem(d + (2 * N_DEV - 2 - s), N_DEV)
            cL = lax.rem(d + 2 + s, N_DEV)
            gemm_tiles(cR, cL, pbR, pbL)

        own_max_box = [jnp.float32(0.0)]

        def rs_add(s, q):
            recv_slot = (s + 1) % 2
            cs = pl.ds(q * QN, QN)
            if s < N_DEV - 2:
                def add_tile(t, _):
                    sl = pl.ds(t * TR, TR)
                    vR = (commR[recv_slot, sl, cs].astype(jnp.float32)
                          + pbR[sl, cs].astype(jnp.float32))
                    commR[recv_slot, sl, cs] = vR.astype(jnp.bfloat16)
                    vL = (commL[recv_slot, sl, cs].astype(jnp.float32)
                          + pbL[sl, cs].astype(jnp.float32))
                    commL[recv_slot, sl, cs] = vL.astype(jnp.bfloat16)
                    return 0
                lax.fori_loop(0, CH // TR, add_tile, 0)
            else:
                def final_tile(t, mx):
                    sl = pl.ds(t * TR, TR)
                    vR = (commR[recv_slot, sl, cs].astype(jnp.float32)
                          + pbR[sl, cs].astype(jnp.float32))
                    ownR = vR.astype(jnp.bfloat16)
                    commR[0, sl, cs] = ownR
                    mx = jnp.maximum(mx, jnp.max(jnp.maximum(
                        ownR.astype(jnp.float32), 0.0)))
                    vL = (commL[recv_slot, sl, cs].astype(jnp.float32)
                          + pbL[sl, cs].astype(jnp.float32))
                    ownL = vL.astype(jnp.bfloat16)
                    commL[0, sl, cs] = ownL
                    return jnp.maximum(mx, jnp.max(jnp.maximum(
                        ownL.astype(jnp.float32), 0.0)))
                own_max_box[0] = lax.fori_loop(
                    0, CH // TR, final_tile, own_max_box[0])

        cur = [rs_rdma(0, 0), rs_rdma(0, 1)]
        rs_gemm(0)
        for s in range(N_DEV - 1):
            for q in (0, 1):
                rR, rL = cur[q]
                rR.wait()
                rL.wait()
                if s <= N_DEV - 3:
                    pl.semaphore_signal(
                        rs_creditR.at[q], inc=1, device_id=(left,),
                        device_id_type=_MESH)
                    pl.semaphore_signal(
                        rs_creditL.at[q], inc=1, device_id=(right,),
                        device_id_type=_MESH)
                rs_add(s, q)
                if s < N_DEV - 2:
                    pl.semaphore_wait(rs_creditR.at[q], 1)
                    pl.semaphore_wait(rs_creditL.at[q], 1)
                    cur[q] = rs_rdma(s + 1, q)
            if s < N_DEV - 2:
                rs_gemm(s + 1)
        own_max = own_max_box[0]

        amax_buf[d, :, :] = jnp.full((8, 128), own_max, dtype=jnp.float32)
        bcasts = []
        for k in range(1, N_DEV):
            j = lax.rem(d + k, N_DEV)
            bc = pltpu.make_async_remote_copy(
                src_ref=amax_buf.at[d],
                dst_ref=amax_buf.at[d],
                send_sem=bc_send_sems.at[k - 1],
                recv_sem=bc_recv_sems.at[k - 1],
                device_id=(j,),
                device_id_type=_MESH,
            )
            bc.start()
            bcasts.append(bc)
        for bc in bcasts:
            bc.wait_recv()
        for bc in bcasts:
            bc.wait_send()
        gmax = jnp.max(amax_buf[...])

        scale = gmax / 127.0
        inv = jnp.where(gmax > 0.0, 127.0 / gmax, 0.0)

        def quant_tile(t, _):
            sl = pl.ds(t * TR, TR)
            vR = commR[0, sl, :].astype(jnp.float32)
            qR = jnp.clip(jnp.round(jnp.maximum(vR, 0.0) * inv), 0.0, 127.0)
            qcommR[0, sl, :] = qR.astype(jnp.int8)
            commR[0, sl, :] = (qR * scale).astype(jnp.bfloat16)
            vL = commL[0, sl, :].astype(jnp.float32)
            qL = jnp.clip(jnp.round(jnp.maximum(vL, 0.0) * inv), 0.0, 127.0)
            qcommL[0, sl, :] = qL.astype(jnp.int8)
            commL[0, sl, :] = (qL * scale).astype(jnp.bfloat16)
            return 0
        lax.fori_loop(0, CH // TR, quant_tile, 0)

        odR = pltpu.make_async_copy(
            commR.at[0], out_ref.at[pl.ds(d * CH, CH), pl.ds(0, HN)], ob_semR)
        odR.start()
        odL = pltpu.make_async_copy(
            commL.at[0], out_ref.at[pl.ds(d * CH, CH), pl.ds(HN, HN)], ob_semL)
        odL.start()
        odR.wait()
        odL.wait()

        pl.semaphore_signal(ag_creditR, inc=1, device_id=(left,),
                            device_id_type=_MESH)
        pl.semaphore_signal(ag_creditL, inc=1, device_id=(right,),
                            device_id_type=_MESH)

        def ag_rdma(s):
            rR = pltpu.make_async_remote_copy(
                src_ref=qcommR.at[s % 2], dst_ref=qcommR.at[(s + 1) % 2],
                send_sem=agR_send.at[s], recv_sem=agR_recv.at[s],
                device_id=(right,), device_id_type=_MESH)
            rL = pltpu.make_async_remote_copy(
                src_ref=qcommL.at[s % 2], dst_ref=qcommL.at[(s + 1) % 2],
                send_sem=agL_send.at[s], recv_sem=agL_recv.at[s],
                device_id=(left,), device_id_type=_MESH)
            rR.start()
            rL.start()
            return rR, rL

        pl.semaphore_wait(ag_creditR, 1)
        pl.semaphore_wait(ag_creditL, 1)
        cur = ag_rdma(0)
        for s in range(N_DEV - 1):
            recv_slot = (s + 1) % 2
            cur[0].wait()
            cur[1].wait()
            if s <= N_DEV - 3:
                pl.semaphore_signal(ag_creditR, inc=1, device_id=(left,),
                                    device_id_type=_MESH)
                pl.semaphore_signal(ag_creditL, inc=1, device_id=(right,),
                                    device_id_type=_MESH)
                pl.semaphore_wait(ag_creditR, 1)
                pl.semaphore_wait(ag_creditL, 1)
                cur = ag_rdma(s + 1)

            def dequant_tile(t, _):
                sl = pl.ds(t * TR, TR)
                commR[recv_slot, sl, :] = (
                    qcommR[recv_slot, sl, :].astype(jnp.float32) * scale
                ).astype(jnp.bfloat16)
                commL[recv_slot, sl, :] = (
                    qcommL[recv_slot, sl, :].astype(jnp.float32) * scale
                ).astype(jnp.bfloat16)
                return 0
            lax.fori_loop(0, CH // TR, dequant_tile, 0)

            cR = lax.rem(d + (2 * N_DEV - 1 - s), N_DEV)
            cL = lax.rem(d + 1 + s, N_DEV)
            stR = pltpu.make_async_copy(
                commR.at[recv_slot],
                out_ref.at[pl.ds(cR * CH, CH), pl.ds(0, HN)], ob_semR)
            stR.start()
            stL = pltpu.make_async_copy(
                commL.at[recv_slot],
                out_ref.at[pl.ds(cL * CH, CH), pl.ds(HN, HN)], ob_semL)
            stL.start()
            stR.wait()
            stL.wait()

    return pl.pallas_call(
        body,
        out_shape=jax.ShapeDtypeStruct((M, N), jnp.bfloat16),
        in_specs=[pl.BlockSpec(memory_space=pltpu.VMEM),
                  pl.BlockSpec(memory_space=pltpu.VMEM)],
        out_specs=pl.BlockSpec(memory_space=pl.ANY),
        scratch_shapes=[
            pltpu.VMEM((2, CH, HN), jnp.bfloat16),
            pltpu.VMEM((2, CH, HN), jnp.bfloat16),
            pltpu.VMEM((2, CH, HN), jnp.int8),
            pltpu.VMEM((2, CH, HN), jnp.int8),
            pltpu.VMEM((CH, HN), jnp.bfloat16),
            pltpu.VMEM((CH, HN), jnp.bfloat16),
            pltpu.VMEM((N_DEV, 8, 128), jnp.float32),
            pltpu.SemaphoreType.DMA,
            pltpu.SemaphoreType.DMA,
            pltpu.SemaphoreType.DMA((N_DEV - 1, 2)),
            pltpu.SemaphoreType.DMA((N_DEV - 1, 2)),
            pltpu.SemaphoreType.DMA((N_DEV - 1, 2)),
            pltpu.SemaphoreType.DMA((N_DEV - 1, 2)),
            pltpu.SemaphoreType.DMA((N_DEV - 1,)),
            pltpu.SemaphoreType.DMA((N_DEV - 1,)),
            pltpu.SemaphoreType.DMA((N_DEV - 1,)),
            pltpu.SemaphoreType.DMA((N_DEV - 1,)),
            pltpu.SemaphoreType.DMA((N_DEV - 1,)),
            pltpu.SemaphoreType.DMA((N_DEV - 1,)),
            pltpu.SemaphoreType.REGULAR((2,)),
            pltpu.SemaphoreType.REGULAR((2,)),
            pltpu.SemaphoreType.REGULAR,
            pltpu.SemaphoreType.REGULAR,
        ],
        compiler_params=pltpu.CompilerParams(
            collective_id=0,
            vmem_limit_bytes=58 * 1024 * 1024,
        ),
    )(x, w_mat)


def kernel(x, w_mat):
    return _fused_gemm_ar(
        x.astype(jnp.bfloat16), w_mat.astype(jnp.bfloat16))


# device time: 585874 ns/iter; 2.4689x vs baseline; 1.0053x over previous
import jax
import jax.numpy as jnp
from jax import lax
from jax.experimental import pallas as pl
from jax.experimental.pallas import tpu as pltpu

N_DEV = 8
M, K, N = 4096, 4096, 8192
CH = M // N_DEV
HN = N // 2
QN = HN // 2
TR = 128

_MESH = pl.DeviceIdType.MESH


def _fused_gemm_ar(x, w_mat):

    def body(x_ref, w_ref, out_ref, commR, commL, qcommR, qcommL,
             pbR, pbL, amax_buf,
             ob_semR, ob_semL, od_semR, od_semL,
             rsR_send, rsR_recv, rsL_send, rsL_recv,
             agR_send, agR_recv, agL_send, agL_recv,
             bc_send_sems, bc_recv_sems,
             rs_creditR, rs_creditL, ag_creditR, ag_creditL):
        d = lax.axis_index("i")
        left = lax.rem(d + (N_DEV - 1), N_DEV)
        right = lax.rem(d + 1, N_DEV)

        cR0 = lax.rem(d + (N_DEV - 1), N_DEV)
        cL0 = lax.rem(d + 1, N_DEV)

        def gemm_tiles(c_right, c_left, dstR, dstL):
            def tile(t, _):
                sl = pl.ds(t * TR, TR)
                xt = x_ref[pl.ds(c_right * CH + t * TR, TR), :]
                dstR[sl, :] = jnp.dot(
                    xt, w_ref[:, 0:HN],
                    preferred_element_type=jnp.float32,
                ).astype(jnp.bfloat16)
                xt2 = x_ref[pl.ds(c_left * CH + t * TR, TR), :]
                dstL[sl, :] = jnp.dot(
                    xt2, w_ref[:, HN:],
                    preferred_element_type=jnp.float32,
                ).astype(jnp.bfloat16)
                return 0
            lax.fori_loop(0, CH // TR, tile, 0)

        gemm_tiles(cR0, cL0, commR.at[0], commL.at[0])

        barrier = pltpu.get_barrier_semaphore()
        pl.semaphore_signal(barrier, inc=1, device_id=(left,),
                            device_id_type=_MESH)
        pl.semaphore_signal(barrier, inc=1, device_id=(right,),
                            device_id_type=_MESH)
        pl.semaphore_wait(barrier, 2)

        def rs_rdma(s, q):
            cs = pl.ds(q * QN, QN)
            rR = pltpu.make_async_remote_copy(
                src_ref=commR.at[s % 2, :, cs],
                dst_ref=commR.at[(s + 1) % 2, :, cs],
                send_sem=rsR_send.at[s, q], recv_sem=rsR_recv.at[s, q],
                device_id=(right,), device_id_type=_MESH)
            rL = pltpu.make_async_remote_copy(
                src_ref=commL.at[s % 2, :, cs],
                dst_ref=commL.at[(s + 1) % 2, :, cs],
                send_sem=rsL_send.at[s, q], recv_sem=rsL_recv.at[s, q],
                device_id=(left,), device_id_type=_MESH)
            rR.start()
            rL.start()
            return rR, rL

        def rs_gemm(s):
            cR = lax.rem(d + (2 * N_DEV - 2 - s), N_DEV)
            cL = lax.rem(d + 2 + s, N_DEV)
            gemm_tiles(cR, cL, pbR, pbL)

        own_max_box = [jnp.float32(0.0)]

        def rs_add(s, q):
            recv_slot = (s + 1) % 2
            cs = pl.ds(q * QN, QN)
            if s < N_DEV - 2:
                def add_tile(t, _):
                    sl = pl.ds(t * TR, TR)
                    vR = (commR[recv_slot, sl, cs].astype(jnp.float32)
                          + pbR[sl, cs].astype(jnp.float32))
                    commR[recv_slot, sl, cs] = vR.astype(jnp.bfloat16)
                    vL = (commL[recv_slot, sl, cs].astype(jnp.float32)
                          + pbL[sl, cs].astype(jnp.float32))
                    commL[recv_slot, sl, cs] = vL.astype(jnp.bfloat16)
                    return 0
                lax.fori_loop(0, CH // TR, add_tile, 0)
            else:
                def final_tile(t, mx):
                    sl = pl.ds(t * TR, TR)
                    vR = (commR[recv_slot, sl, cs].astype(jnp.float32)
                          + pbR[sl, cs].astype(jnp.float32))
                    ownR = vR.astype(jnp.bfloat16)
                    commR[0, sl, cs] = ownR
                    mx = jnp.maximum(mx, jnp.max(jnp.maximum(
                        ownR.astype(jnp.float32), 0.0)))
                    vL = (commL[recv_slot, sl, cs].astype(jnp.float32)
                          + pbL[sl, cs].astype(jnp.float32))
                    ownL = vL.astype(jnp.bfloat16)
                    commL[0, sl, cs] = ownL
                    return jnp.maximum(mx, jnp.max(jnp.maximum(
                        ownL.astype(jnp.float32), 0.0)))
                own_max_box[0] = lax.fori_loop(
                    0, CH // TR, final_tile, own_max_box[0])

        cur = [rs_rdma(0, 0), rs_rdma(0, 1)]
        rs_gemm(0)
        for s in range(N_DEV - 1):
            for q in (0, 1):
                rR, rL = cur[q]
                rR.wait()
                rL.wait()
                if s <= N_DEV - 3:
                    pl.semaphore_signal(
                        rs_creditR.at[q], inc=1, device_id=(left,),
                        device_id_type=_MESH)
                    pl.semaphore_signal(
                        rs_creditL.at[q], inc=1, device_id=(right,),
                        device_id_type=_MESH)
                rs_add(s, q)
                if s < N_DEV - 2:
                    pl.semaphore_wait(rs_creditR.at[q], 1)
                    pl.semaphore_wait(rs_creditL.at[q], 1)
                    cur[q] = rs_rdma(s + 1, q)
            if s < N_DEV - 2:
                rs_gemm(s + 1)
        own_max = own_max_box[0]

        amax_buf[d, :, :] = jnp.full((8, 128), own_max, dtype=jnp.float32)
        bcasts = []
        for k in range(1, N_DEV):
            j = lax.rem(d + k, N_DEV)
            bc = pltpu.make_async_remote_copy(
                src_ref=amax_buf.at[d],
                dst_ref=amax_buf.at[d],
                send_sem=bc_send_sems.at[k - 1],
                recv_sem=bc_recv_sems.at[k - 1],
                device_id=(j,),
                device_id_type=_MESH,
            )
            bc.start()
            bcasts.append(bc)
        for bc in bcasts:
            bc.wait_recv()
        for bc in bcasts:
            bc.wait_send()
        gmax = jnp.max(amax_buf[...])

        scale = gmax / 127.0
        inv = jnp.where(gmax > 0.0, 127.0 / gmax, 0.0)

        def quant_tile(t, _):
            sl = pl.ds(t * TR, TR)
            vR = commR[0, sl, :].astype(jnp.float32)
            qR = jnp.clip(jnp.round(jnp.maximum(vR, 0.0) * inv), 0.0, 127.0)
            qcommR[0, sl, :] = qR.astype(jnp.int8)
            commR[0, sl, :] = (qR * scale).astype(jnp.bfloat16)
            vL = commL[0, sl, :].astype(jnp.float32)
            qL = jnp.clip(jnp.round(jnp.maximum(vL, 0.0) * inv), 0.0, 127.0)
            qcommL[0, sl, :] = qL.astype(jnp.int8)
            commL[0, sl, :] = (qL * scale).astype(jnp.bfloat16)
            return 0
        lax.fori_loop(0, CH // TR, quant_tile, 0)

        odR = pltpu.make_async_copy(
            commR.at[0], out_ref.at[pl.ds(d * CH, CH), pl.ds(0, HN)], od_semR)
        odR.start()
        odL = pltpu.make_async_copy(
            commL.at[0], out_ref.at[pl.ds(d * CH, CH), pl.ds(HN, HN)], od_semL)
        odL.start()

        pl.semaphore_signal(ag_creditR, inc=1, device_id=(left,),
                            device_id_type=_MESH)
        pl.semaphore_signal(ag_creditL, inc=1, device_id=(right,),
                            device_id_type=_MESH)

        def ag_rdma(s):
            rR = pltpu.make_async_remote_copy(
                src_ref=qcommR.at[s % 2], dst_ref=qcommR.at[(s + 1) % 2],
                send_sem=agR_send.at[s], recv_sem=agR_recv.at[s],
                device_id=(right,), device_id_type=_MESH)
            rL = pltpu.make_async_remote_copy(
                src_ref=qcommL.at[s % 2], dst_ref=qcommL.at[(s + 1) % 2],
                send_sem=agL_send.at[s], recv_sem=agL_recv.at[s],
                device_id=(left,), device_id_type=_MESH)
            rR.start()
            rL.start()
            return rR, rL

        pl.semaphore_wait(ag_creditR, 1)
        pl.semaphore_wait(ag_creditL, 1)
        cur = ag_rdma(0)
        for s in range(N_DEV - 1):
            recv_slot = (s + 1) % 2
            cur[0].wait()
            cur[1].wait()
            if s <= N_DEV - 3:
                pl.semaphore_signal(ag_creditR, inc=1, device_id=(left,),
                                    device_id_type=_MESH)
                pl.semaphore_signal(ag_creditL, inc=1, device_id=(right,),
                                    device_id_type=_MESH)
                pl.semaphore_wait(ag_creditR, 1)
                pl.semaphore_wait(ag_creditL, 1)
                cur = ag_rdma(s + 1)
            if s == 1:
                odR.wait()
                odL.wait()

            def dequant_tile(t, _):
                sl = pl.ds(t * TR, TR)
                commR[recv_slot, sl, :] = (
                    qcommR[recv_slot, sl, :].astype(jnp.float32) * scale
                ).astype(jnp.bfloat16)
                commL[recv_slot, sl, :] = (
                    qcommL[recv_slot, sl, :].astype(jnp.float32) * scale
                ).astype(jnp.bfloat16)
                return 0
            lax.fori_loop(0, CH // TR, dequant_tile, 0)

            cR = lax.rem(d + (2 * N_DEV - 1 - s), N_DEV)
            cL = lax.rem(d + 1 + s, N_DEV)
            stR = pltpu.make_async_copy(
                commR.at[recv_slot],
                out_ref.at[pl.ds(cR * CH, CH), pl.ds(0, HN)], ob_semR)
            stR.start()
            stL = pltpu.make_async_copy(
                commL.at[recv_slot],
                out_ref.at[pl.ds(cL * CH, CH), pl.ds(HN, HN)], ob_semL)
            stL.start()
            stR.wait()
            stL.wait()

    return pl.pallas_call(
        body,
        out_shape=jax.ShapeDtypeStruct((M, N), jnp.bfloat16),
        in_specs=[pl.BlockSpec(memory_space=pltpu.VMEM),
                  pl.BlockSpec(memory_space=pltpu.VMEM)],
        out_specs=pl.BlockSpec(memory_space=pl.ANY),
        scratch_shapes=[
            pltpu.VMEM((2, CH, HN), jnp.bfloat16),
            pltpu.VMEM((2, CH, HN), jnp.bfloat16),
            pltpu.VMEM((2, CH, HN), jnp.int8),
            pltpu.VMEM((2, CH, HN), jnp.int8),
            pltpu.VMEM((CH, HN), jnp.bfloat16),
            pltpu.VMEM((CH, HN), jnp.bfloat16),
            pltpu.VMEM((N_DEV, 8, 128), jnp.float32),
            pltpu.SemaphoreType.DMA,
            pltpu.SemaphoreType.DMA,
            pltpu.SemaphoreType.DMA,
            pltpu.SemaphoreType.DMA,
            pltpu.SemaphoreType.DMA((N_DEV - 1, 2)),
            pltpu.SemaphoreType.DMA((N_DEV - 1, 2)),
            pltpu.SemaphoreType.DMA((N_DEV - 1, 2)),
            pltpu.SemaphoreType.DMA((N_DEV - 1, 2)),
            pltpu.SemaphoreType.DMA((N_DEV - 1,)),
            pltpu.SemaphoreType.DMA((N_DEV - 1,)),
            pltpu.SemaphoreType.DMA((N_DEV - 1,)),
            pltpu.SemaphoreType.DMA((N_DEV - 1,)),
            pltpu.SemaphoreType.DMA((N_DEV - 1,)),
            pltpu.SemaphoreType.DMA((N_DEV - 1,)),
            pltpu.SemaphoreType.REGULAR((2,)),
            pltpu.SemaphoreType.REGULAR((2,)),
            pltpu.SemaphoreType.REGULAR,
            pltpu.SemaphoreType.REGULAR,
        ],
        compiler_params=pltpu.CompilerParams(
            collective_id=0,
            vmem_limit_bytes=58 * 1024 * 1024,
        ),
    )(x, w_mat)


def kernel(x, w_mat):
    return _fused_gemm_ar(
        x.astype(jnp.bfloat16), w_mat.astype(jnp.bfloat16))
